# baseline (speedup 1.0000x reference)
import functools

import jax
import jax.numpy as jnp
from jax import lax
from jax.experimental import pallas as pl
from jax.experimental.pallas import tpu as pltpu
from jax.experimental.pallas import tpu_sc as plsc

VOCAB_ROWS = 1002
D = 50
B, L = 4096, 256
N = B * L
TW = VOCAB_ROWS * D

NUM_CORES = 2
NUM_SUBCORES = 16
NW = NUM_CORES * NUM_SUBCORES
BPW = N // NW

CHUNK = 512
NCHUNK = BPW // CHUNK
IBLK = 8
GRP = 16
NGRP = CHUNK // GRP

_mesh = plsc.VectorSubcoreMesh(core_axis_name="c", subcore_axis_name="s")


@functools.partial(
    pl.kernel,
    mesh=_mesh,
    compiler_params=pltpu.CompilerParams(
        use_tc_tiling_on_sc=False, needs_layout_passes=False
    ),
    out_type=jax.ShapeDtypeStruct((N * D,), jnp.float32),
    scratch_types=[
        pltpu.VMEM((IBLK * CHUNK,), jnp.int32),
        pltpu.VMEM((TW,), jnp.float32),
        pltpu.VMEM((CHUNK * D,), jnp.float32),
        pltpu.VMEM((CHUNK * D,), jnp.float32),
        pltpu.SemaphoreType.DMA,
        pltpu.SemaphoreType.DMA,
    ],
)
def _embed_lookup(idx_hbm, table_hbm, out_hbm, idx_v, table_v,
                  packed0, packed1, sem0, sem1):
    wid = lax.axis_index("s") * NUM_CORES + lax.axis_index("c")
    base = wid * BPW
    bufs = (packed0, packed1)
    sems = (sem0, sem1)

    pltpu.sync_copy(table_hbm, table_v)

    def outer(co, carry):
        @pl.when(lax.rem(co, IBLK // 2) == 0)
        def _stage_idx():
            ioff = pl.multiple_of(base + (co * 2) * CHUNK, IBLK * CHUNK)
            pltpu.sync_copy(idx_hbm.at[pl.ds(ioff, IBLK * CHUNK)], idx_v)

        ib = lax.rem(co, IBLK // 2)
        for b in range(2):
            ci = co * 2 + b
            off = pl.multiple_of(base + ci * CHUNK, CHUNK)
            loc = pl.multiple_of((ib * 2 + b) * CHUNK, CHUNK)
            buf = bufs[b]
            sem = sems[b]

            @pl.when(co > 0)
            def _drain():
                pltpu.make_async_copy(
                    buf, out_hbm.at[pl.ds(0, CHUNK * D)], sem
                ).wait()

            def grp_body(g, carry2):
                tbase = loc + g * GRP
                goff = g * (GRP * D)
                mod = lax.iota(jnp.int32, 16)
                div = jnp.zeros((16,), jnp.int32)
                for k in range(D):
                    tok = plsc.load_gather(idx_v, [div + tbase])
                    val = plsc.load_gather(table_v, [tok * D + mod])
                    buf[pl.ds(goff + k * 16, 16)] = val
                    if k + 1 < D:
                        m2 = mod + 16
                        wrap = m2 >= D
                        mod = jnp.where(wrap, m2 - D, m2)
                        div = jnp.where(wrap, div + 1, div)
                return carry2

            lax.fori_loop(0, NGRP, grp_body, 0)

            pltpu.async_copy(buf, out_hbm.at[pl.ds(off * D, CHUNK * D)], sem)
        return carry

    lax.fori_loop(0, NCHUNK // 2, outer, 0)

    for b in range(2):
        pltpu.make_async_copy(
            bufs[b], out_hbm.at[pl.ds(0, CHUNK * D)], sems[b]
        ).wait()


def kernel(seqs, att_mask, word_embedding):
    del att_mask
    idx = seqs.reshape(N).astype(jnp.int32)
    out = _embed_lookup(idx, word_embedding.reshape(TW))
    return out.reshape(B, L, D)


# speedup vs baseline: 1.4322x; 1.4322x over previous
import functools

import jax
import jax.numpy as jnp
from jax import lax
from jax.experimental import pallas as pl
from jax.experimental.pallas import tpu as pltpu
from jax.experimental.pallas import tpu_sc as plsc

VOCAB_ROWS = 1002
D = 50
B, L = 4096, 256
N = B * L
TW = VOCAB_ROWS * D

NUM_CORES = 2
NUM_SUBCORES = 16
NW = NUM_CORES * NUM_SUBCORES
BPW = N // NW

CHUNK = 512
NCHUNK = BPW // CHUNK
IBLK = 8
GRP = 16
NGRP = CHUNK // GRP

_mesh = plsc.VectorSubcoreMesh(core_axis_name="c", subcore_axis_name="s")


@functools.partial(
    pl.kernel,
    mesh=_mesh,
    compiler_params=pltpu.CompilerParams(
        use_tc_tiling_on_sc=False, needs_layout_passes=False
    ),
    out_type=jax.ShapeDtypeStruct((N * D,), jnp.float32),
    scratch_types=[
        pltpu.VMEM((IBLK * CHUNK,), jnp.int32),
        pltpu.VMEM((TW,), jnp.float32),
        pltpu.VMEM((CHUNK * D,), jnp.float32),
        pltpu.VMEM((CHUNK * D,), jnp.float32),
        pltpu.SemaphoreType.DMA,
        pltpu.SemaphoreType.DMA,
    ],
)
def _embed_lookup(idx_hbm, table_hbm, out_hbm, idx_v, table_v,
                  packed0, packed1, sem0, sem1):
    wid = lax.axis_index("s") * NUM_CORES + lax.axis_index("c")
    base = wid * BPW
    bufs = (packed0, packed1)
    sems = (sem0, sem1)

    pltpu.sync_copy(table_hbm, table_v)

    def outer(co, carry):
        @pl.when(lax.rem(co, IBLK // 2) == 0)
        def _stage_idx():
            ioff = pl.multiple_of(base + (co * 2) * CHUNK, IBLK * CHUNK)
            pltpu.sync_copy(idx_hbm.at[pl.ds(ioff, IBLK * CHUNK)], idx_v)

        ib = lax.rem(co, IBLK // 2)
        for b in range(2):
            ci = co * 2 + b
            off = pl.multiple_of(base + ci * CHUNK, CHUNK)
            loc = pl.multiple_of((ib * 2 + b) * CHUNK, CHUNK)
            buf = bufs[b]
            sem = sems[b]

            @pl.when(co > 0)
            def _drain():
                pltpu.make_async_copy(
                    buf, out_hbm.at[pl.ds(0, CHUNK * D)], sem
                ).wait()

            @plsc.parallel_loop(0, NGRP, unroll=2)
            def grp_body(g):
                tbase = loc + g * GRP
                goff = g * (GRP * D)
                mod = lax.iota(jnp.int32, 16)
                div = jnp.zeros((16,), jnp.int32)
                for k in range(D):
                    tok = plsc.load_gather(idx_v, [div + tbase])
                    val = plsc.load_gather(table_v, [tok * D + mod])
                    buf[pl.ds(goff + k * 16, 16)] = val
                    if k + 1 < D:
                        m2 = mod + 16
                        wrap = m2 >= D
                        mod = jnp.where(wrap, m2 - D, m2)
                        div = jnp.where(wrap, div + 1, div)

            pltpu.async_copy(buf, out_hbm.at[pl.ds(off * D, CHUNK * D)], sem)
        return carry

    lax.fori_loop(0, NCHUNK // 2, outer, 0)

    for b in range(2):
        pltpu.make_async_copy(
            bufs[b], out_hbm.at[pl.ds(0, CHUNK * D)], sems[b]
        ).wait()


def kernel(seqs, att_mask, word_embedding):
    del att_mask
    idx = seqs.reshape(N).astype(jnp.int32)
    out = _embed_lookup(idx, word_embedding.reshape(TW))
    return out.reshape(B, L, D)


# speedup vs baseline: 1.5012x; 1.0482x over previous
import functools

import jax
import jax.numpy as jnp
import numpy as np
from jax import lax
from jax.experimental import pallas as pl
from jax.experimental.pallas import tpu as pltpu
from jax.experimental.pallas import tpu_sc as plsc

VOCAB_ROWS = 1002
D = 50
DP = 64
B, L = 4096, 256
N = B * L

NUM_CORES = 2
NUM_SUBCORES = 16
NW = NUM_CORES * NUM_SUBCORES
BPW = N // NW

GROUP = 128
CHUNK = 512
G = CHUNK // GROUP
NCHUNK = BPW // CHUNK
NBLK = CHUNK // 8
NPAT = (8 * D) // 16

_mesh = plsc.VectorSubcoreMesh(core_axis_name="c", subcore_axis_name="s")


@functools.partial(
    pl.kernel,
    mesh=_mesh,
    compiler_params=pltpu.CompilerParams(
        use_tc_tiling_on_sc=False, needs_layout_passes=False
    ),
    out_type=jax.ShapeDtypeStruct((N * D,), jnp.float32),
    scratch_types=[
        pltpu.VMEM((2 * G, GROUP), jnp.int32),
        pltpu.VMEM((2 * NPAT * 16,), jnp.int32),
        pltpu.VMEM((CHUNK, DP), jnp.float32),
        pltpu.VMEM((CHUNK, DP), jnp.float32),
        pltpu.VMEM((CHUNK * D,), jnp.float32),
        pltpu.VMEM((CHUNK * D,), jnp.float32),
        pltpu.SemaphoreType.DMA,
        pltpu.SemaphoreType.DMA,
        pltpu.SemaphoreType.DMA,
    ],
)
def _embed_lookup(idx_hbm, table_hbm, pool_hbm, out_hbm, idx_v, pool_v,
                  rows0, rows1, packed0, packed1, gsem, osem0, osem1):
    wid = lax.axis_index("s") * NUM_CORES + lax.axis_index("c")
    base = wid * BPW
    rows = (rows0, rows1)
    packed = (packed0, packed1)
    osems = (osem0, osem1)

    pltpu.sync_copy(pool_hbm, pool_v)

    def outer(co, carry):
        irow = pl.multiple_of((base + co * 2 * CHUNK) // GROUP, 2 * G)
        pltpu.sync_copy(idx_hbm.at[pl.ds(irow, 2 * G)], idx_v)

        for b in range(2):
            ci = co * 2 + b
            off = pl.multiple_of(base + ci * CHUNK, CHUNK)
            rbuf = rows[b]
            pbuf = packed[b]
            osem = osems[b]

            copies = [
                pltpu.async_copy(
                    table_hbm.at[idx_v.at[b * G + j]],
                    rbuf.at[pl.ds(j * GROUP, GROUP)],
                    gsem,
                )
                for j in range(G)
            ]
            for c in copies:
                c.wait()

            @pl.when(co > 0)
            def _drain():
                pltpu.make_async_copy(
                    pbuf, out_hbm.at[pl.ds(0, CHUNK * D)], osem
                ).wait()

            for k in range(NPAT):
                rp = pool_v[pl.ds(k * 16, 16)]
                cp = pool_v[pl.ds(NPAT * 16 + k * 16, 16)]

                @plsc.parallel_loop(0, NBLK, unroll=4)
                def _blk(blk):
                    val = plsc.load_gather(rbuf, [rp + blk * 8, cp])
                    pbuf[pl.ds(blk * (8 * D) + k * 16, 16)] = val

            pltpu.async_copy(pbuf, out_hbm.at[pl.ds(off * D, CHUNK * D)], osem)
        return carry

    lax.fori_loop(0, NCHUNK // 2, outer, 0)

    for b in range(2):
        pltpu.make_async_copy(
            packed[b], out_hbm.at[pl.ds(0, CHUNK * D)], osems[b]
        ).wait()


def _address_pool() -> jax.Array:
    o = np.arange(NPAT * 16)
    return jnp.asarray(
        np.concatenate([o // D, o % D]).astype(np.int32)
    )


def kernel(seqs, att_mask, word_embedding):
    del att_mask
    idx2d = seqs.reshape(N // GROUP, GROUP).astype(jnp.int32)
    table_p = jnp.pad(word_embedding, ((0, 0), (0, DP - D)))
    out = _embed_lookup(idx2d, table_p, _address_pool())
    return out.reshape(B, L, D)


# speedup vs baseline: 1.5152x; 1.0093x over previous
import functools

import jax
import jax.numpy as jnp
import numpy as np
from jax import lax
from jax.experimental import pallas as pl
from jax.experimental.pallas import tpu as pltpu
from jax.experimental.pallas import tpu_sc as plsc

VOCAB_ROWS = 1002
D = 50
DP = 64
B, L = 4096, 256
N = B * L

NUM_CORES = 2
NUM_SUBCORES = 16
NW = NUM_CORES * NUM_SUBCORES
BPW = N // NW

GROUP = 128
CHUNK = 512
G = CHUNK // GROUP
NCHUNK = BPW // CHUNK
NBLK = CHUNK // 8
NPAT = (8 * D) // 16

_mesh = plsc.VectorSubcoreMesh(core_axis_name="c", subcore_axis_name="s")


@functools.partial(
    pl.kernel,
    mesh=_mesh,
    compiler_params=pltpu.CompilerParams(
        use_tc_tiling_on_sc=False, needs_layout_passes=False
    ),
    out_type=jax.ShapeDtypeStruct((N * D,), jnp.float32),
    scratch_types=[
        pltpu.VMEM((2 * G, GROUP), jnp.int32),
        pltpu.VMEM((CHUNK, DP), jnp.float32),
        pltpu.VMEM((CHUNK, DP), jnp.float32),
        pltpu.VMEM((CHUNK * D,), jnp.float32),
        pltpu.VMEM((CHUNK * D,), jnp.float32),
        pltpu.SemaphoreType.DMA,
        pltpu.SemaphoreType.DMA,
        pltpu.SemaphoreType.DMA,
    ],
)
def _embed_lookup(idx_hbm, table_hbm, out_hbm, idx_v,
                  rows0, rows1, packed0, packed1, gsem, osem0, osem1):
    wid = lax.axis_index("s") * NUM_CORES + lax.axis_index("c")
    base = wid * BPW
    rows = (rows0, rows1)
    packed = (packed0, packed1)
    osems = (osem0, osem1)

    def outer(co, carry):
        irow = pl.multiple_of((base + co * 2 * CHUNK) // GROUP, 2 * G)
        pltpu.sync_copy(idx_hbm.at[pl.ds(irow, 2 * G)], idx_v)

        for b in range(2):
            ci = co * 2 + b
            off = pl.multiple_of(base + ci * CHUNK, CHUNK)
            rbuf = rows[b]
            pbuf = packed[b]
            osem = osems[b]

            copies = [
                pltpu.async_copy(
                    table_hbm.at[idx_v.at[b * G + j]],
                    rbuf.at[pl.ds(j * GROUP, GROUP)],
                    gsem,
                )
                for j in range(G)
            ]
            for c in copies:
                c.wait()

            @pl.when(co > 0)
            def _drain():
                pltpu.make_async_copy(
                    pbuf, out_hbm.at[pl.ds(0, CHUNK * D)], osem
                ).wait()

            @plsc.parallel_loop(0, NBLK, unroll=2)
            def _blk(blk):
                rbase = blk * 8
                dbase = blk * (8 * D)
                for rr in range(8):
                    for c in (0, 16, 32, 34):
                        pbuf[pl.ds(dbase + rr * D + c, 16)] = (
                            rbuf[rbase + rr, pl.ds(c, 16)]
                        )

            pltpu.async_copy(pbuf, out_hbm.at[pl.ds(off * D, CHUNK * D)], osem)
        return carry

    lax.fori_loop(0, NCHUNK // 2, outer, 0)

    for b in range(2):
        pltpu.make_async_copy(
            packed[b], out_hbm.at[pl.ds(0, CHUNK * D)], osems[b]
        ).wait()


def kernel(seqs, att_mask, word_embedding):
    del att_mask
    idx2d = seqs.reshape(N // GROUP, GROUP).astype(jnp.int32)
    table_p = jnp.pad(word_embedding, ((0, 0), (0, DP - D)))
    out = _embed_lookup(idx2d, table_p)
    return out.reshape(B, L, D)


# speedup vs baseline: 1.8270x; 1.2058x over previous
import functools

import jax
import jax.numpy as jnp
from jax import lax
from jax.experimental import pallas as pl
from jax.experimental.pallas import tpu as pltpu
from jax.experimental.pallas import tpu_sc as plsc

VOCAB_ROWS = 1002
D = 50
DP = 64
B, L = 4096, 256
N = B * L

NUM_CORES = 2
NUM_SUBCORES = 16
NW = NUM_CORES * NUM_SUBCORES
BPW = N // NW

GROUP = 128
CHUNK = 512
G = CHUNK // GROUP
NCHUNK = BPW // CHUNK
NBLK = CHUNK // 8

_mesh = plsc.VectorSubcoreMesh(core_axis_name="c", subcore_axis_name="s")


@functools.partial(
    pl.kernel,
    mesh=_mesh,
    compiler_params=pltpu.CompilerParams(
        use_tc_tiling_on_sc=False, needs_layout_passes=False
    ),
    out_type=jax.ShapeDtypeStruct((N * D,), jnp.float32),
    scratch_types=[
        pltpu.VMEM_SHARED((VOCAB_ROWS, DP), jnp.float32),
        pltpu.VMEM((2 * G, GROUP), jnp.int32),
        pltpu.VMEM((2 * G, GROUP), jnp.int32),
        pltpu.VMEM((CHUNK, DP), jnp.float32),
        pltpu.VMEM((CHUNK, DP), jnp.float32),
        pltpu.VMEM((CHUNK * D,), jnp.float32),
        pltpu.VMEM((CHUNK * D,), jnp.float32),
        pltpu.SemaphoreType.DMA,
        pltpu.SemaphoreType.DMA,
        pltpu.SemaphoreType.DMA,
        pltpu.SemaphoreType.DMA,
    ],
)
def _embed_lookup(idx_hbm, table_hbm, out_hbm, table_sp, idx0, idx1,
                  rows0, rows1, packed0, packed1,
                  gsem0, gsem1, osem0, osem1):
    wid = lax.axis_index("s") * NUM_CORES + lax.axis_index("c")
    base = wid * BPW
    idxb = (idx0, idx1)
    rows = (rows0, rows1)
    packed = (packed0, packed1)
    gsems = (gsem0, gsem1)
    osems = (osem0, osem1)

    @pl.when(lax.axis_index("s") == 0)
    def _stage_table():
        pltpu.sync_copy(table_hbm, table_sp)

    plsc.subcore_barrier()

    def _stage_idx(co, buf):
        irow = pl.multiple_of((base + co * 2 * CHUNK) // GROUP, 2 * G)
        pltpu.sync_copy(idx_hbm.at[pl.ds(irow, 2 * G)], buf)

    def _fire_gathers(p, ib, slot):
        for j in range(G):
            pltpu.async_copy(
                table_sp.at[ib.at[slot * G + j]],
                rows[p].at[pl.ds(j * GROUP, GROUP)],
                gsems[p],
            )

    def _drain_gathers(p):
        pltpu.make_async_copy(
            table_sp.at[pl.ds(0, CHUNK)], rows[p], gsems[p]
        ).wait()

    _stage_idx(0, idxb[0])
    _fire_gathers(0, idxb[0], 0)

    def outer(cq, carry):
        for hb in range(2):
            co = cq * 2 + hb

            @pl.when(co + 1 < NCHUNK // 2)
            def _():
                _stage_idx(co + 1, idxb[(hb + 1) % 2])

            for b in range(2):
                ci = co * 2 + b
                off = pl.multiple_of(base + ci * CHUNK, CHUNK)
                rbuf = rows[b]
                pbuf = packed[b]
                osem = osems[b]

                _drain_gathers(b)

                nxt_ib = idxb[hb] if b == 0 else idxb[(hb + 1) % 2]
                nxt_slot = (b + 1) % 2

                @pl.when(ci + 1 < NCHUNK)
                def _next():
                    _fire_gathers((b + 1) % 2, nxt_ib, nxt_slot)

                @pl.when(ci > 1)
                def _drain_out():
                    pltpu.make_async_copy(
                        pbuf, out_hbm.at[pl.ds(0, CHUNK * D)], osem
                    ).wait()

                @plsc.parallel_loop(0, NBLK, unroll=2)
                def _blk(blk):
                    rbase = blk * 8
                    dbase = blk * (8 * D)
                    for rr in range(8):
                        for c in (0, 16, 32, 34):
                            pbuf[pl.ds(dbase + rr * D + c, 16)] = (
                                rbuf[rbase + rr, pl.ds(c, 16)]
                            )

                pltpu.async_copy(
                    pbuf, out_hbm.at[pl.ds(off * D, CHUNK * D)], osem
                )
        return carry

    lax.fori_loop(0, NCHUNK // 4, outer, 0)

    for b in range(2):
        pltpu.make_async_copy(
            packed[b], out_hbm.at[pl.ds(0, CHUNK * D)], osems[b]
        ).wait()


def kernel(seqs, att_mask, word_embedding):
    del att_mask
    idx2d = seqs.reshape(N // GROUP, GROUP).astype(jnp.int32)
    table_p = jnp.pad(word_embedding, ((0, 0), (0, DP - D)))
    out = _embed_lookup(idx2d, table_p)
    return out.reshape(B, L, D)
